# baseline (speedup 1.0000x reference)
import functools

import jax
import jax.numpy as jnp
from jax import lax
from jax.experimental import pallas as pl
from jax.experimental.pallas import tpu as pltpu
from jax.experimental.pallas import tpu_sc as plsc

N_NODES = 10000
N_EDGES = 160000
DIM = 16
NUM_FEATURES = 128
NUM_CLASSES = 10
NUM_GRAPHS = 64

_EDGE_BLK = 1280


def _full(shape):
    return pl.BlockSpec(shape, lambda *a: tuple(0 for _ in shape))


def _msg_big_body(attr_ref, xs_ref, wet_ref, bet_ref, l1_ref, b1_ref,
                  l2p_ref, b2p_ref, ea_ref, msg_ref):
    ea = attr_ref[...] @ wet_ref[...] + bet_ref[...]
    ea_ref[...] = ea
    h = jnp.maximum(ea @ l1_ref[...] + b1_ref[...], 0.0)
    w2 = h @ l2p_ref[...] + b2p_ref[...]
    xs = xs_ref[...]
    cols = []
    for o in range(DIM):
        cols.append(jnp.sum(
            xs * w2[:, o * NUM_FEATURES:(o + 1) * NUM_FEATURES],
            axis=1, keepdims=True))
    msg_ref[...] = jnp.concatenate(cols, axis=1)


def _msg_big(edge_attr, xs, wet, bet, l1, b1, l2p, b2p):
    nblk = N_EDGES // _EDGE_BLK
    return pl.pallas_call(
        _msg_big_body,
        grid=(nblk,),
        in_specs=[
            pl.BlockSpec((_EDGE_BLK, DIM), lambda i: (i, 0)),
            pl.BlockSpec((_EDGE_BLK, NUM_FEATURES), lambda i: (i, 0)),
            _full((DIM, DIM)), _full((1, DIM)),
            _full((DIM, NUM_FEATURES)), _full((1, NUM_FEATURES)),
            _full((NUM_FEATURES, DIM * NUM_FEATURES)),
            _full((1, DIM * NUM_FEATURES)),
        ],
        out_specs=[
            pl.BlockSpec((_EDGE_BLK, DIM), lambda i: (i, 0)),
            pl.BlockSpec((_EDGE_BLK, DIM), lambda i: (i, 0)),
        ],
        out_shape=[
            jax.ShapeDtypeStruct((N_EDGES, DIM), jnp.float32),
            jax.ShapeDtypeStruct((N_EDGES, DIM), jnp.float32),
        ],
    )(edge_attr, xs, wet, bet, l1, b1, l2p, b2p)


def _msg_small_body(ea_ref, xs_ref, l1_ref, b1_ref, l2p_ref, b2p_ref, msg_ref):
    h = jnp.maximum(ea_ref[...] @ l1_ref[...] + b1_ref[...], 0.0)
    w2 = h @ l2p_ref[...] + b2p_ref[...]
    xs = xs_ref[...][:, :DIM]
    cols = []
    for o in range(DIM):
        cols.append(jnp.sum(
            xs * w2[:, o * DIM:(o + 1) * DIM], axis=1, keepdims=True))
    msg_ref[...] = jnp.concatenate(cols, axis=1)


def _msg_small(ea, xs, l1, b1, l2p, b2p):
    nblk = N_EDGES // _EDGE_BLK
    return pl.pallas_call(
        _msg_small_body,
        grid=(nblk,),
        in_specs=[
            pl.BlockSpec((_EDGE_BLK, DIM), lambda i: (i, 0)),
            pl.BlockSpec((_EDGE_BLK, NUM_FEATURES), lambda i: (i, 0)),
            _full((DIM, NUM_FEATURES)), _full((1, NUM_FEATURES)),
            _full((NUM_FEATURES, DIM * DIM)), _full((1, DIM * DIM)),
        ],
        out_specs=pl.BlockSpec((_EDGE_BLK, DIM), lambda i: (i, 0)),
        out_shape=jax.ShapeDtypeStruct((N_EDGES, DIM), jnp.float32),
    )(ea, xs, l1, b1, l2p, b2p)


def _bn_body(aggr_ref, aggr2_ref, hin_ref, root_ref, bias_ref, g_ref, b_ref,
             out_ref, *, cin, residual):
    hu = hin_ref[...][:, :cin]
    z = (aggr_ref[...] + aggr2_ref[...] + hu @ root_ref[...] + bias_ref[...])
    m = jnp.mean(z, axis=0, keepdims=True)
    v = jnp.mean((z - m) ** 2, axis=0, keepdims=True)
    out = jnp.maximum((z - m) * lax.rsqrt(v + 1e-5) * g_ref[...] + b_ref[...],
                      0.0)
    if residual:
        out = out + hu
    out_ref[...] = jnp.concatenate(
        [out, jnp.zeros((N_NODES, NUM_FEATURES - DIM), jnp.float32)], axis=1)


def _bn_update(aggr, aggr2, hin, root, bias, gamma, beta, residual):
    cin = root.shape[0]
    return pl.pallas_call(
        functools.partial(_bn_body, cin=cin, residual=residual),
        in_specs=[
            _full((N_NODES, DIM)), _full((N_NODES, DIM)),
            _full((N_NODES, NUM_FEATURES)),
            _full((cin, DIM)), _full((1, DIM)),
            _full((1, DIM)), _full((1, DIM)),
        ],
        out_specs=_full((N_NODES, NUM_FEATURES)),
        out_shape=jax.ShapeDtypeStruct((N_NODES, NUM_FEATURES), jnp.float32),
    )(aggr, aggr2, hin, root, bias, gamma, beta)


def _head_body(h_ref, batch_ref, w0_ref, b0_ref, w1_ref, b1_ref,
               w2_ref, b2_ref, out_ref):
    h = h_ref[...][:, :DIM]
    b = batch_ref[...]
    rows = []
    for g in range(NUM_GRAPHS):
        rows.append(jnp.max(jnp.where(b == g, h, -jnp.inf),
                            axis=0, keepdims=True))
    pooled = jnp.concatenate(rows, axis=0)
    g0 = jnp.maximum(pooled @ w0_ref[...] + b0_ref[...], 0.0)
    g1 = jnp.maximum(g0 @ w1_ref[...] + b1_ref[...], 0.0)
    logits = g1 @ w2_ref[...] + b2_ref[...]
    out_ref[...] = 1.0 / (1.0 + jnp.exp(-logits))


def _head(h, batch2, p):
    return pl.pallas_call(
        _head_body,
        in_specs=[
            _full((N_NODES, NUM_FEATURES)), _full((N_NODES, 1)),
            _full((DIM, DIM)), _full((1, DIM)),
            _full((DIM, DIM)), _full((1, DIM)),
            _full((DIM, NUM_CLASSES)), _full((1, NUM_CLASSES)),
        ],
        out_specs=_full((NUM_GRAPHS, NUM_CLASSES)),
        out_shape=jax.ShapeDtypeStruct((NUM_GRAPHS, NUM_CLASSES), jnp.float32),
    )(h, batch2,
      p["lin0"]["W"], p["lin0"]["b"].reshape(1, -1),
      p["lin1"]["W"], p["lin1"]["b"].reshape(1, -1),
      p["lin_out"]["W"], p["lin_out"]["b"].reshape(1, -1))


_NC = 2
_NS = 16
_NW = _NC * _NS
_CHUNK = 128
_NCHUNKS = N_EDGES // _CHUNK
_ROWS_PER_TILE = 632
_NROW_PAD = _NS * _ROWS_PER_TILE
_NCHUNKS_PAD = 1280
_E_PAD = _NCHUNKS_PAD * _CHUNK
_GROUPS_PER_TILE = _NCHUNKS_PAD // 8 // _NW


def _sc_gather(table, src_pad):
    n_feat = table.shape[1]
    mesh = plsc.VectorSubcoreMesh(core_axis_name="c", subcore_axis_name="s")

    @functools.partial(
        pl.kernel, mesh=mesh,
        out_type=jax.ShapeDtypeStruct((_E_PAD, n_feat), jnp.float32),
        scratch_types=(
            [pltpu.VMEM((_CHUNK,), jnp.int32) for _ in range(4)]
            + [pltpu.VMEM((_CHUNK, n_feat), jnp.float32) for _ in range(4)]
            + [pltpu.SemaphoreType.DMA, pltpu.SemaphoreType.DMA]
        ),
    )
    def gk(table_hbm, src_hbm, out_hbm, *scratch):
        idxs = scratch[0:4]
        rows = scratch[4:8]
        semg, semw = scratch[8:]
        wid = lax.axis_index("s") * _NC + lax.axis_index("c")
        base = wid * (_NCHUNKS_PAD // _NW)

        def body(q, carry):
            c0 = base + q * 4
            gws = []
            for b in range(4):
                pltpu.sync_copy(
                    src_hbm.at[pl.ds((c0 + b) * _CHUNK, _CHUNK)], idxs[b])
                gws.append(pltpu.async_copy(table_hbm.at[idxs[b]],
                                            rows[b], semg))
            wws = []
            for b in range(4):
                gws[b].wait()
                wws.append(pltpu.async_copy(
                    rows[b], out_hbm.at[pl.ds((c0 + b) * _CHUNK, _CHUNK)],
                    semw))
            for w in wws:
                w.wait()
            return carry

        lax.fori_loop(0, _NCHUNKS_PAD // _NW // 4, body, 0)

    return gk(table, src_pad)


_NELEM = _NROW_PAD * DIM
_EST = _NELEM // _NS
_SUB = 16


def _sc_scatter_add(msg_flat, dstel, zeros_flat):
    mesh = plsc.VectorSubcoreMesh(core_axis_name="c", subcore_axis_name="s")

    @functools.partial(
        pl.kernel, mesh=mesh,
        out_type=jax.ShapeDtypeStruct((2 * _NELEM,), jnp.float32),
        scratch_types=[
            pltpu.VMEM((_SUB, _CHUNK), jnp.int32),
            pltpu.VMEM((_CHUNK,), jnp.int32),
            pltpu.VMEM((_CHUNK * DIM,), jnp.float32),
            pltpu.VMEM_SHARED((_NELEM,), jnp.float32),
        ],
    )
    def sk(msg_hbm, dst_hbm, zero_hbm, out_hbm, idx_v, idx1, msg_v, acc):
        cid = lax.axis_index("c")
        sid = lax.axis_index("s")
        pltpu.sync_copy(zero_hbm.at[pl.ds(sid * _EST, _EST)],
                        acc.at[pl.ds(sid * _EST, _EST)])
        plsc.subcore_barrier()
        wid = cid * _NS + sid

        def body(g, carry):
            for j in range(8):
                c = g * 8 + j
                pltpu.sync_copy(dst_hbm.at[pl.ds(c * _SUB, _SUB)], idx_v)
                pltpu.sync_copy(msg_hbm.at[pl.ds(c * _CHUNK * DIM,
                                                 _CHUNK * DIM)], msg_v)
                for s in range(_SUB):
                    for k in range(_CHUNK // 16):
                        idx1[pl.ds(k * 16, 16)] = idx_v[s, pl.ds(k * 16, 16)]
                    pltpu.sync_copy(msg_v.at[pl.ds(s * _CHUNK, _CHUNK)],
                                    acc.at[idx1], add=True)
            return carry

        lax.fori_loop(wid * _GROUPS_PER_TILE, (wid + 1) * _GROUPS_PER_TILE,
                      body, 0)
        plsc.subcore_barrier()
        pltpu.sync_copy(acc.at[pl.ds(sid * _EST, _EST)],
                        out_hbm.at[pl.ds(cid * _NELEM + sid * _EST, _EST)])

    return sk(msg_flat, dstel, zeros_flat)


def _gather_rows(table, src):
    return _sc_gather(table, src)


def _scatter_add_parts(msg, dstel, zeros_flat):
    msg_flat = jnp.concatenate(
        [msg, jnp.zeros((_E_PAD - N_EDGES, DIM), jnp.float32)],
        axis=0).reshape(-1)
    parts = _sc_scatter_add(msg_flat, dstel, zeros_flat)
    p0 = parts[:_NELEM].reshape(_NROW_PAD, DIM)
    p1 = parts[_NELEM:].reshape(_NROW_PAD, DIM)
    return p0[:N_NODES], p1[:N_NODES]


def _perm_l2(w, cin):
    return w.reshape(NUM_FEATURES, cin, DIM).transpose(0, 2, 1).reshape(
        NUM_FEATURES, DIM * cin)


def _perm_b2(b, cin):
    return b.reshape(cin, DIM).T.reshape(1, DIM * cin)


def kernel(x, edge_index, edge_attr, batch, params):
    src = edge_index[0]
    dst_pad = jnp.concatenate(
        [edge_index[1], jnp.zeros((_E_PAD - N_EDGES,), jnp.int32)])
    dstel = (dst_pad[:, None] * DIM
             + jnp.arange(DIM, dtype=jnp.int32)[None, :]).reshape(
                 _E_PAD * DIM // _CHUNK, _CHUNK)
    zeros = jnp.zeros((_NELEM,), jnp.float32)
    p = params
    et = p["edge_transform"]
    nn0 = p["conv_in"]["nn"]
    l2p0 = _perm_l2(nn0["l2"]["W"], NUM_FEATURES)
    b2p0 = _perm_b2(nn0["l2"]["b"], NUM_FEATURES)

    src_pad = jnp.concatenate(
        [src, jnp.zeros((_E_PAD - N_EDGES,), jnp.int32)])
    xs0 = _gather_rows(x, src_pad)
    ea, msg0 = _msg_big(edge_attr, xs0,
                        et["W"], et["b"].reshape(1, -1),
                        nn0["l1"]["W"], nn0["l1"]["b"].reshape(1, -1),
                        l2p0, b2p0)
    a0, b0 = _scatter_add_parts(msg0, dstel, zeros)
    hn = _bn_update(a0, b0, x, p["conv_in"]["root"],
                    p["conv_in"]["bias"].reshape(1, -1),
                    p["bns"][0]["gamma"].reshape(1, -1),
                    p["bns"][0]["beta"].reshape(1, -1), residual=False)

    for i in range(4):
        cv = p["convs"][i]
        l2p = _perm_l2(cv["nn"]["l2"]["W"], DIM)
        b2p = _perm_b2(cv["nn"]["l2"]["b"], DIM)
        xs = _gather_rows(hn, src_pad)
        msg = _msg_small(ea, xs, cv["nn"]["l1"]["W"],
                         cv["nn"]["l1"]["b"].reshape(1, -1), l2p, b2p)
        aa, bb = _scatter_add_parts(msg, dstel, zeros)
        hn = _bn_update(aa, bb, hn, cv["root"], cv["bias"].reshape(1, -1),
                        p["bns"][i + 1]["gamma"].reshape(1, -1),
                        p["bns"][i + 1]["beta"].reshape(1, -1), residual=True)

    return _head(hn, batch.reshape(N_NODES, 1), p)


# speedup vs baseline: 1.1089x; 1.1089x over previous
import functools

import jax
import jax.numpy as jnp
from jax import lax
from jax.experimental import pallas as pl
from jax.experimental.pallas import tpu as pltpu
from jax.experimental.pallas import tpu_sc as plsc

N_NODES = 10000
N_EDGES = 160000
DIM = 16
NUM_FEATURES = 128
NUM_CLASSES = 10
NUM_GRAPHS = 64

_EDGE_BLK = 1000


def _full(shape):
    return pl.BlockSpec(shape, lambda *a: tuple(0 for _ in shape))


def _msg_big_body(attr_ref, xs_ref, wet_ref, bet_ref, l1_ref, b1_ref,
                  l2p_ref, b2p_ref, ea_ref, msg_ref):
    ea = attr_ref[...] @ wet_ref[...] + bet_ref[...]
    ea_ref[...] = ea
    h = jnp.maximum(ea @ l1_ref[...] + b1_ref[...], 0.0)
    w2 = h @ l2p_ref[...] + b2p_ref[...]
    xs = xs_ref[...]
    cols = []
    for o in range(DIM):
        cols.append(jnp.sum(
            xs * w2[:, o * NUM_FEATURES:(o + 1) * NUM_FEATURES],
            axis=1, keepdims=True))
    msg_ref[...] = jnp.concatenate(cols, axis=1)


def _msg_big(edge_attr, xs, wet, bet, l1, b1, l2p, b2p):
    nblk = N_EDGES // _EDGE_BLK
    return pl.pallas_call(
        _msg_big_body,
        grid=(nblk,),
        in_specs=[
            pl.BlockSpec((_EDGE_BLK, DIM), lambda i: (i, 0)),
            pl.BlockSpec((_EDGE_BLK, NUM_FEATURES), lambda i: (i, 0)),
            _full((DIM, DIM)), _full((1, DIM)),
            _full((DIM, NUM_FEATURES)), _full((1, NUM_FEATURES)),
            _full((NUM_FEATURES, DIM * NUM_FEATURES)),
            _full((1, DIM * NUM_FEATURES)),
        ],
        out_specs=[
            pl.BlockSpec((_EDGE_BLK, DIM), lambda i: (i, 0)),
            pl.BlockSpec((_EDGE_BLK, DIM), lambda i: (i, 0)),
        ],
        out_shape=[
            jax.ShapeDtypeStruct((N_EDGES, DIM), jnp.float32),
            jax.ShapeDtypeStruct((N_EDGES, DIM), jnp.float32),
        ],
    )(edge_attr, xs, wet, bet, l1, b1, l2p, b2p)


def _msg_small_body(ea_ref, xs_ref, l1_ref, b1_ref, l2p_ref, b2p_ref, msg_ref):
    h = jnp.maximum(ea_ref[...] @ l1_ref[...] + b1_ref[...], 0.0)
    w2 = h @ l2p_ref[...] + b2p_ref[...]
    xs = xs_ref[...][:, :DIM]
    cols = []
    for o in range(DIM):
        cols.append(jnp.sum(
            xs * w2[:, o * DIM:(o + 1) * DIM], axis=1, keepdims=True))
    msg_ref[...] = jnp.concatenate(cols, axis=1)


def _msg_small(ea, xs, l1, b1, l2p, b2p):
    nblk = N_EDGES // _EDGE_BLK
    return pl.pallas_call(
        _msg_small_body,
        grid=(nblk,),
        in_specs=[
            pl.BlockSpec((_EDGE_BLK, DIM), lambda i: (i, 0)),
            pl.BlockSpec((_EDGE_BLK, NUM_FEATURES), lambda i: (i, 0)),
            _full((DIM, NUM_FEATURES)), _full((1, NUM_FEATURES)),
            _full((NUM_FEATURES, DIM * DIM)), _full((1, DIM * DIM)),
        ],
        out_specs=pl.BlockSpec((_EDGE_BLK, DIM), lambda i: (i, 0)),
        out_shape=jax.ShapeDtypeStruct((N_EDGES, DIM), jnp.float32),
    )(ea, xs, l1, b1, l2p, b2p)


def _bn_body(aggr_ref, aggr2_ref, hin_ref, root_ref, bias_ref, g_ref, b_ref,
             out_ref, *, cin, residual):
    hu = hin_ref[...][:, :cin]
    z = (aggr_ref[...] + aggr2_ref[...] + hu @ root_ref[...] + bias_ref[...])
    m = jnp.mean(z, axis=0, keepdims=True)
    v = jnp.mean((z - m) ** 2, axis=0, keepdims=True)
    out = jnp.maximum((z - m) * lax.rsqrt(v + 1e-5) * g_ref[...] + b_ref[...],
                      0.0)
    if residual:
        out = out + hu
    out_ref[...] = jnp.concatenate(
        [out, jnp.zeros((N_NODES, NUM_FEATURES - DIM), jnp.float32)], axis=1)


def _bn_update(aggr, aggr2, hin, root, bias, gamma, beta, residual):
    cin = root.shape[0]
    return pl.pallas_call(
        functools.partial(_bn_body, cin=cin, residual=residual),
        in_specs=[
            _full((N_NODES, DIM)), _full((N_NODES, DIM)),
            _full((N_NODES, NUM_FEATURES)),
            _full((cin, DIM)), _full((1, DIM)),
            _full((1, DIM)), _full((1, DIM)),
        ],
        out_specs=_full((N_NODES, NUM_FEATURES)),
        out_shape=jax.ShapeDtypeStruct((N_NODES, NUM_FEATURES), jnp.float32),
    )(aggr, aggr2, hin, root, bias, gamma, beta)


def _head_body(h_ref, batch_ref, w0_ref, b0_ref, w1_ref, b1_ref,
               w2_ref, b2_ref, out_ref):
    h = h_ref[...][:, :DIM]
    b = batch_ref[...]
    rows = []
    for g in range(NUM_GRAPHS):
        rows.append(jnp.max(jnp.where(b == g, h, -jnp.inf),
                            axis=0, keepdims=True))
    pooled = jnp.concatenate(rows, axis=0)
    g0 = jnp.maximum(pooled @ w0_ref[...] + b0_ref[...], 0.0)
    g1 = jnp.maximum(g0 @ w1_ref[...] + b1_ref[...], 0.0)
    logits = g1 @ w2_ref[...] + b2_ref[...]
    out_ref[...] = 1.0 / (1.0 + jnp.exp(-logits))


def _head(h, batch2, p):
    return pl.pallas_call(
        _head_body,
        in_specs=[
            _full((N_NODES, NUM_FEATURES)), _full((N_NODES, 1)),
            _full((DIM, DIM)), _full((1, DIM)),
            _full((DIM, DIM)), _full((1, DIM)),
            _full((DIM, NUM_CLASSES)), _full((1, NUM_CLASSES)),
        ],
        out_specs=_full((NUM_GRAPHS, NUM_CLASSES)),
        out_shape=jax.ShapeDtypeStruct((NUM_GRAPHS, NUM_CLASSES), jnp.float32),
    )(h, batch2,
      p["lin0"]["W"], p["lin0"]["b"].reshape(1, -1),
      p["lin1"]["W"], p["lin1"]["b"].reshape(1, -1),
      p["lin_out"]["W"], p["lin_out"]["b"].reshape(1, -1))


_NC = 2
_NS = 16
_NW = _NC * _NS
_CHUNK = 128
_NCHUNKS = N_EDGES // _CHUNK
_ROWS_PER_TILE = 632
_NROW_PAD = _NS * _ROWS_PER_TILE
_NCHUNKS_PAD = 1280
_E_PAD = _NCHUNKS_PAD * _CHUNK
_GROUPS_PER_TILE = _NCHUNKS_PAD // 8 // _NW


def _sc_gather(table, src):
    n_feat = table.shape[1]
    mesh = plsc.VectorSubcoreMesh(core_axis_name="c", subcore_axis_name="s")

    @functools.partial(
        pl.kernel, mesh=mesh,
        out_type=jax.ShapeDtypeStruct((N_EDGES, n_feat), jnp.float32),
        scratch_types=[
            pltpu.VMEM((_CHUNK,), jnp.int32),
            pltpu.VMEM((_CHUNK, n_feat), jnp.float32),
            pltpu.SemaphoreType.DMA,
        ],
    )
    def gk(table_hbm, src_hbm, out_hbm, idx_v, rows_v, sem):
        wid = lax.axis_index("s") * _NC + lax.axis_index("c")
        lo = wid * _NCHUNKS // _NW
        hi = (wid + 1) * _NCHUNKS // _NW

        def body(c, carry):
            pltpu.sync_copy(src_hbm.at[pl.ds(c * _CHUNK, _CHUNK)], idx_v)
            pltpu.async_copy(table_hbm.at[idx_v], rows_v, sem).wait()
            pltpu.sync_copy(rows_v, out_hbm.at[pl.ds(c * _CHUNK, _CHUNK)])
            return carry

        lax.fori_loop(lo, hi, body, 0)

    return gk(table, src)


_NELEM = _NROW_PAD * DIM
_EST = _NELEM // _NS
_SUB = 16


def _sc_scatter_add(msg_flat, dstel, zeros_flat):
    mesh = plsc.VectorSubcoreMesh(core_axis_name="c", subcore_axis_name="s")

    @functools.partial(
        pl.kernel, mesh=mesh,
        out_type=jax.ShapeDtypeStruct((2 * _NELEM,), jnp.float32),
        scratch_types=(
            [pltpu.VMEM((_SUB, _CHUNK), jnp.int32)]
            + [pltpu.VMEM((_CHUNK,), jnp.int32) for _ in range(_SUB)]
            + [pltpu.VMEM((_CHUNK * DIM,), jnp.float32),
               pltpu.VMEM_SHARED((_NELEM,), jnp.float32),
               pltpu.SemaphoreType.DMA]
        ),
    )
    def sk(msg_hbm, dst_hbm, zero_hbm, out_hbm, *scratch):
        idx_v = scratch[0]
        idx1 = scratch[1:1 + _SUB]
        msg_v, acc, sem = scratch[1 + _SUB:]
        cid = lax.axis_index("c")
        sid = lax.axis_index("s")
        pltpu.sync_copy(zero_hbm.at[pl.ds(sid * _EST, _EST)],
                        acc.at[pl.ds(sid * _EST, _EST)])
        plsc.subcore_barrier()
        wid = cid * _NS + sid

        def body(g, carry):
            for j in range(8):
                c = g * 8 + j
                pltpu.sync_copy(dst_hbm.at[pl.ds(c * _SUB, _SUB)], idx_v)
                pltpu.sync_copy(msg_hbm.at[pl.ds(c * _CHUNK * DIM,
                                                 _CHUNK * DIM)], msg_v)
                for s in range(_SUB):
                    for k in range(_CHUNK // 16):
                        idx1[s][pl.ds(k * 16, 16)] = idx_v[s, pl.ds(k * 16, 16)]
                waits = []
                for s in range(_SUB):
                    waits.append(pltpu.async_copy(
                        msg_v.at[pl.ds(s * _CHUNK, _CHUNK)],
                        acc.at[idx1[s]], sem, add=True))
                for w in waits:
                    w.wait()
            return carry

        lax.fori_loop(wid * _GROUPS_PER_TILE, (wid + 1) * _GROUPS_PER_TILE,
                      body, 0)
        plsc.subcore_barrier()
        pltpu.sync_copy(acc.at[pl.ds(sid * _EST, _EST)],
                        out_hbm.at[pl.ds(cid * _NELEM + sid * _EST, _EST)])

    return sk(msg_flat, dstel, zeros_flat)


def _gather_rows(table, src):
    return _sc_gather(table, src)


def _scatter_add_parts(msg, dstel, zeros_flat):
    msg_flat = jnp.concatenate(
        [msg, jnp.zeros((_E_PAD - N_EDGES, DIM), jnp.float32)],
        axis=0).reshape(-1)
    parts = _sc_scatter_add(msg_flat, dstel, zeros_flat)
    p0 = parts[:_NELEM].reshape(_NROW_PAD, DIM)
    p1 = parts[_NELEM:].reshape(_NROW_PAD, DIM)
    return p0[:N_NODES], p1[:N_NODES]


def _perm_l2(w, cin):
    return w.reshape(NUM_FEATURES, cin, DIM).transpose(0, 2, 1).reshape(
        NUM_FEATURES, DIM * cin)


def _perm_b2(b, cin):
    return b.reshape(cin, DIM).T.reshape(1, DIM * cin)


def kernel(x, edge_index, edge_attr, batch, params):
    src = edge_index[0]
    dst_pad = jnp.concatenate(
        [edge_index[1], jnp.zeros((_E_PAD - N_EDGES,), jnp.int32)])
    dstel = (dst_pad[:, None] * DIM
             + jnp.arange(DIM, dtype=jnp.int32)[None, :]).reshape(
                 _E_PAD * DIM // _CHUNK, _CHUNK)
    zeros = jnp.zeros((_NELEM,), jnp.float32)
    p = params
    et = p["edge_transform"]
    nn0 = p["conv_in"]["nn"]
    l2p0 = _perm_l2(nn0["l2"]["W"], NUM_FEATURES)
    b2p0 = _perm_b2(nn0["l2"]["b"], NUM_FEATURES)

    xs0 = _gather_rows(x, src)
    ea, msg0 = _msg_big(edge_attr, xs0,
                        et["W"], et["b"].reshape(1, -1),
                        nn0["l1"]["W"], nn0["l1"]["b"].reshape(1, -1),
                        l2p0, b2p0)
    a0, b0 = _scatter_add_parts(msg0, dstel, zeros)
    hn = _bn_update(a0, b0, x, p["conv_in"]["root"],
                    p["conv_in"]["bias"].reshape(1, -1),
                    p["bns"][0]["gamma"].reshape(1, -1),
                    p["bns"][0]["beta"].reshape(1, -1), residual=False)

    for i in range(4):
        cv = p["convs"][i]
        l2p = _perm_l2(cv["nn"]["l2"]["W"], DIM)
        b2p = _perm_b2(cv["nn"]["l2"]["b"], DIM)
        xs = _gather_rows(hn, src)
        msg = _msg_small(ea, xs, cv["nn"]["l1"]["W"],
                         cv["nn"]["l1"]["b"].reshape(1, -1), l2p, b2p)
        aa, bb = _scatter_add_parts(msg, dstel, zeros)
        hn = _bn_update(aa, bb, hn, cv["root"], cv["bias"].reshape(1, -1),
                        p["bns"][i + 1]["gamma"].reshape(1, -1),
                        p["bns"][i + 1]["beta"].reshape(1, -1), residual=True)

    return _head(hn, batch.reshape(N_NODES, 1), p)
